# Initial kernel scaffold; baseline (speedup 1.0000x reference)
import functools

import jax
import jax.numpy as jnp
from jax import lax
from jax.experimental import pallas as pl
from jax.experimental.pallas import tpu as pltpu
from jax.experimental.pallas import tpu_sc as plsc

C = 8
N = 32768
TIME = 512
BATCH = 8192

NC = 2
NS = 16
NW = NC * NS
ROWS = C * BATCH
RPW = ROWS // NW
WPC = NW // C
QPW = RPW // 8
WLEN = (RPW - 8) + TIME

_mesh = plsc.VectorSubcoreMesh(core_axis_name="c", subcore_axis_name="s")


@functools.partial(
    pl.kernel,
    out_type=jax.ShapeDtypeStruct((C, BATCH, TIME), jnp.float32),
    mesh=_mesh,
    scratch_types=[
        pltpu.VMEM((8, WLEN), jnp.float32),
        pltpu.SemaphoreType.DMA,
        pltpu.SemaphoreType.DMA,
    ],
)
def _hankel_sc(xs_hbm, out_hbm, win_v, in_sem, out_sem):
    wid = lax.axis_index("s") * NC + lax.axis_index("c")
    chan = wid // WPC
    b0 = (wid % WPC) * RPW
    base = chan * N + b0

    for j in range(8):
        pltpu.async_copy(xs_hbm.at[j, pl.ds(base, WLEN)], win_v.at[j], in_sem)
    for j in range(8):
        pltpu.make_async_copy(
            xs_hbm.at[j, pl.ds(base, WLEN)], win_v.at[j], in_sem
        ).wait()

    def issue(q, carry):
        off = 8 * q
        row = b0 + off
        for j in range(8):
            pltpu.async_copy(
                win_v.at[j, pl.ds(off, TIME)],
                out_hbm.at[chan, row + j],
                out_sem,
            )
        return carry

    lax.fori_loop(0, QPW, issue, 0)

    def drain(q, carry):
        for j in range(8):
            pltpu.make_async_copy(
                win_v.at[0, pl.ds(0, TIME)],
                out_hbm.at[chan, b0],
                out_sem,
            ).wait()
        return carry

    lax.fori_loop(0, QPW, drain, 0)


def kernel(x):
    xf = x.reshape(-1)
    xp = jnp.concatenate([xf, jnp.zeros((8,), jnp.float32)])
    xs = jnp.stack(
        [lax.dynamic_slice(xp, (j,), (C * N,)) for j in range(8)]
    )
    return _hankel_sc(xs)


# speedup vs baseline: 67.2014x; 67.2014x over previous
import functools

import jax
import jax.numpy as jnp
from jax import lax
from jax.experimental import pallas as pl
from jax.experimental.pallas import tpu as pltpu
from jax.experimental.pallas import tpu_sc as plsc

C = 8
N = 32768
TIME = 512
BATCH = 8192

NC = 2
NS = 16
NW = NC * NS
ROWS = C * BATCH
RPW = ROWS // NW
WPC = NW // C
QPW = RPW // 8
WLEN = (RPW - 8) + TIME

_mesh = plsc.VectorSubcoreMesh(core_axis_name="c", subcore_axis_name="s")


@functools.partial(
    pl.kernel,
    out_type=jax.ShapeDtypeStruct((C, BATCH, TIME), jnp.float32),
    mesh=_mesh,
    scratch_types=[
        pltpu.VMEM((8, WLEN), jnp.float32),
        pltpu.SemaphoreType.DMA,
        pltpu.SemaphoreType.DMA,
    ],
    compiler_params=pltpu.CompilerParams(use_tc_tiling_on_sc=False),
)
def _hankel_sc(xs_hbm, out_hbm, win_v, in_sem, out_sem):
    wid = lax.axis_index("s") * NC + lax.axis_index("c")
    chan = wid // WPC
    b0 = (wid % WPC) * RPW
    base = chan * N + b0

    for j in range(8):
        pltpu.async_copy(xs_hbm.at[j, pl.ds(base, WLEN)], win_v.at[j], in_sem)
    for j in range(8):
        pltpu.make_async_copy(
            xs_hbm.at[j, pl.ds(base, WLEN)], win_v.at[j], in_sem
        ).wait()

    def issue(q, carry):
        off = 8 * q
        row = b0 + off
        for j in range(8):
            pltpu.async_copy(
                win_v.at[j, pl.ds(off, TIME)],
                out_hbm.at[chan, row + j],
                out_sem,
            )
        return carry

    lax.fori_loop(0, QPW, issue, 0)

    def drain(q, carry):
        for j in range(8):
            pltpu.make_async_copy(
                win_v.at[0, pl.ds(0, TIME)],
                out_hbm.at[chan, b0],
                out_sem,
            ).wait()
        return carry

    lax.fori_loop(0, QPW, drain, 0)


def kernel(x):
    xf = x.reshape(-1)
    xp = jnp.concatenate([xf, jnp.zeros((8,), jnp.float32)])
    xs = jnp.stack(
        [lax.dynamic_slice(xp, (j,), (C * N,)) for j in range(8)]
    )
    return _hankel_sc(xs)


# speedup vs baseline: 67.5080x; 1.0046x over previous
import functools

import jax
import jax.numpy as jnp
from jax import lax
from jax.experimental import pallas as pl
from jax.experimental.pallas import tpu as pltpu
from jax.experimental.pallas import tpu_sc as plsc

C = 8
N = 32768
TIME = 512
BATCH = 8192

NC = 2
NS = 16
NW = NC * NS
ROWS = C * BATCH
RPW = ROWS // NW
WPC = NW // C
SH = 32
QPW = RPW // SH
WLEN = (RPW - SH) + TIME

_mesh = plsc.VectorSubcoreMesh(core_axis_name="c", subcore_axis_name="s")


@functools.partial(
    pl.kernel,
    out_type=jax.ShapeDtypeStruct((C, BATCH, TIME), jnp.float32),
    mesh=_mesh,
    scratch_types=[
        pltpu.VMEM((SH, WLEN), jnp.float32),
        pltpu.SemaphoreType.DMA,
        pltpu.SemaphoreType.DMA,
    ],
    compiler_params=pltpu.CompilerParams(use_tc_tiling_on_sc=False),
)
def _hankel_sc(xs_hbm, out_hbm, win_v, in_sem, out_sem):
    wid = lax.axis_index("s") * NC + lax.axis_index("c")
    chan = wid // WPC
    b0 = (wid % WPC) * RPW
    base = chan * N + b0

    for j in range(SH):
        pltpu.async_copy(
            xs_hbm.at[j % 8, pl.ds(base + 8 * (j // 8), WLEN)],
            win_v.at[j],
            in_sem,
        )
    for j in range(SH):
        pltpu.make_async_copy(
            xs_hbm.at[0, pl.ds(0, WLEN)], win_v.at[0], in_sem
        ).wait()

    for q in range(QPW):
        pltpu.async_copy(
            win_v.at[pl.ds(0, SH), pl.ds(SH * q, TIME)],
            out_hbm.at[chan, pl.ds(b0 + SH * q, SH)],
            out_sem,
        )
    for q in range(QPW):
        pltpu.make_async_copy(
            win_v.at[pl.ds(0, SH), pl.ds(0, TIME)],
            out_hbm.at[chan, pl.ds(b0, SH)],
            out_sem,
        ).wait()


def kernel(x):
    xf = x.reshape(-1)
    xp = jnp.concatenate([xf, jnp.zeros((8,), jnp.float32)])
    xs = jnp.stack(
        [lax.dynamic_slice(xp, (j,), (C * N,)) for j in range(8)]
    )
    return _hankel_sc(xs)
